# Initial kernel scaffold; baseline (speedup 1.0000x reference)
import jax
import jax.numpy as jnp
from jax.experimental import pallas as pl


def kernel(input1, input2, table, W, b):
    raise NotImplementedError("write your pallas kernel here")


# speedup vs baseline: 2.5590x; 2.5590x over previous
import functools

import jax
import jax.numpy as jnp
from jax import lax
from jax.experimental import pallas as pl
from jax.experimental.pallas import tpu as pltpu
from jax.experimental.pallas import tpu_sc as plsc

_VOCAB = 1000001
_B = 16384
_L = 50
_D = 300
_C = 2 * _L
_BM = 2048

_NC = 2
_NS = 16
_NW = _NC * _NS
_BPW = _B // _NW
_G = 128
_NG = _C * _BPW // _G
_K = 8


def _mm_body(t_ref, w_ref, o_ref):
    o_ref[...] = jnp.dot(t_ref[...], w_ref[...],
                         preferred_element_type=jnp.float32)


def _score_table(table, wt):
    nb = pl.cdiv(_VOCAB, _BM)
    return pl.pallas_call(
        _mm_body,
        grid=(nb,),
        in_specs=[
            pl.BlockSpec((_BM, _D), lambda i: (i, 0)),
            pl.BlockSpec((_D, _C), lambda i: (0, 0)),
        ],
        out_specs=pl.BlockSpec((_BM, _C), lambda i: (i, 0)),
        out_shape=jax.ShapeDtypeStruct((_VOCAB, _C), jnp.float32),
    )(table, wt)


def _gather_reduce(p_flat, idx, bvec):
    mesh = plsc.VectorSubcoreMesh(core_axis_name="c", subcore_axis_name="s")

    @functools.partial(
        pl.kernel,
        mesh=mesh,
        out_type=jax.ShapeDtypeStruct((_B,), jnp.float32),
        scratch_types=[
            pltpu.VMEM((_NG, _G), jnp.int32),
            pltpu.VMEM((_NG * _G,), jnp.float32),
            pltpu.VMEM((16,), jnp.float32),
            pltpu.VMEM((_BPW,), jnp.float32),
            pltpu.SemaphoreType.DMA,
        ],
    )
    def k(p_hbm, idx_hbm, b_hbm, out_hbm, idx_v, g_v, b_v, out_v, sem):
        wid = lax.axis_index("s") * _NC + lax.axis_index("c")
        pltpu.sync_copy(idx_hbm.at[wid], idx_v)
        pltpu.sync_copy(b_hbm, b_v)

        def fire_group(gi, carry):
            cps = []
            for j in range(_K):
                row = gi * _K + j
                cps.append(pltpu.async_copy(
                    p_hbm.at[idx_v.at[row]],
                    g_v.at[pl.ds(row * _G, _G)],
                    sem,
                ))
            for cp in cps:
                cp.wait()
            return carry

        lax.fori_loop(0, _NG // _K, fire_group, 0)

        def col(c, carry):
            def red(l, acc):
                return acc + g_v[pl.ds(l * _BPW + c * 16, 16)]

            acc = lax.fori_loop(0, _C, red, b_v[...])
            out_v[pl.ds(c * 16, 16)] = 1.0 / (1.0 + jnp.exp(-acc))
            return carry

        lax.fori_loop(0, _BPW // 16, col, 0)
        pltpu.sync_copy(out_v, out_hbm.at[pl.ds(wid * _BPW, _BPW)])

    return k(p_flat, idx, bvec)


def kernel(input1, input2, table, W, b):
    wt = W.reshape(_C, _D).T.astype(jnp.float32)
    P = _score_table(table.astype(jnp.float32), wt)

    pos = jnp.arange(_L, dtype=jnp.int32)
    idx1 = input1 * _C + pos[None, :]
    idx2 = input2 * _C + (_L + pos)[None, :]
    idx_all = jnp.concatenate([idx1, idx2], axis=1)
    idx_r = (idx_all.reshape(_NW, _BPW, _C)
             .transpose(0, 2, 1)
             .reshape(_NW, _NG, _G))

    bvec = jnp.broadcast_to(b.astype(jnp.float32), (16,))
    out = _gather_reduce(P.reshape(-1), idx_r, bvec)
    return out.reshape(_B, 1)


# speedup vs baseline: 3.4260x; 1.3388x over previous
import functools

import jax
import jax.numpy as jnp
from jax import lax
from jax.experimental import pallas as pl
from jax.experimental.pallas import tpu as pltpu
from jax.experimental.pallas import tpu_sc as plsc

_VOCAB = 1000001
_VPAD = 1000008
_B = 16384
_L = 50
_D = 300
_C = 2 * _L
_CP = 128
_BM = 2048

_NC = 2
_NS = 16
_NW = _NC * _NS
_BPW = _B // _NW
_G = 128
_NG = _C * _BPW // _G
_K = 8


def _mm_body(t_ref, w_ref, o_ref):
    o_ref[...] = jnp.dot(t_ref[...], w_ref[...],
                         preferred_element_type=jnp.float32)


def _score_table(table, wt):
    nb = pl.cdiv(_VPAD, _BM)
    return pl.pallas_call(
        _mm_body,
        grid=(nb,),
        in_specs=[
            pl.BlockSpec((_BM, _D), lambda i: (i, 0)),
            pl.BlockSpec((_D, _CP), lambda i: (0, 0)),
        ],
        out_specs=pl.BlockSpec((_BM, _CP), lambda i: (i, 0)),
        out_shape=jax.ShapeDtypeStruct((_VPAD, _CP), jnp.float32),
    )(table, wt)


def _gather_reduce(p_flat, idx, bvec):
    mesh = plsc.VectorSubcoreMesh(core_axis_name="c", subcore_axis_name="s")

    @functools.partial(
        pl.kernel,
        mesh=mesh,
        out_type=jax.ShapeDtypeStruct((_B,), jnp.float32),
        scratch_types=[
            pltpu.VMEM((_NG, _G), jnp.int32),
            pltpu.VMEM((_NG * _G,), jnp.float32),
            pltpu.VMEM((16,), jnp.float32),
            pltpu.VMEM((_BPW,), jnp.float32),
            pltpu.SemaphoreType.DMA,
        ],
    )
    def k(p_hbm, idx_hbm, b_hbm, out_hbm, idx_v, g_v, b_v, out_v, sem):
        wid = lax.axis_index("s") * _NC + lax.axis_index("c")
        pltpu.sync_copy(idx_hbm.at[wid], idx_v)
        pltpu.sync_copy(b_hbm, b_v)

        def fire_group(gi, carry):
            cps = []
            for j in range(_K):
                row = gi * _K + j
                cps.append(pltpu.async_copy(
                    p_hbm.at[idx_v.at[row]],
                    g_v.at[pl.ds(row * _G, _G)],
                    sem,
                ))
            for cp in cps:
                cp.wait()
            return carry

        lax.fori_loop(0, _NG // _K, fire_group, 0)

        def col(c, carry):
            def red(l, acc):
                return acc + g_v[pl.ds(l * _BPW + c * 16, 16)]

            acc = lax.fori_loop(0, _C, red, b_v[...])
            out_v[pl.ds(c * 16, 16)] = 1.0 / (1.0 + jnp.exp(-acc))
            return carry

        lax.fori_loop(0, _BPW // 16, col, 0)
        pltpu.sync_copy(out_v, out_hbm.at[pl.ds(wid * _BPW, _BPW)])

    return k(p_flat, idx, bvec)


def kernel(input1, input2, table, W, b):
    wt = W.reshape(_C, _D).T.astype(jnp.float32)
    wt = jnp.pad(wt, ((0, 0), (0, _CP - _C)))
    P = _score_table(table.astype(jnp.float32), wt)

    pos = jnp.arange(_L, dtype=jnp.int32)
    idx1 = input1 * _CP + pos[None, :]
    idx2 = input2 * _CP + (_L + pos)[None, :]
    idx_all = jnp.concatenate([idx1, idx2], axis=1)
    idx_r = (idx_all.reshape(_NW, _BPW, _C)
             .transpose(0, 2, 1)
             .reshape(_NW, _NG, _G))

    bvec = jnp.broadcast_to(b.astype(jnp.float32), (16,))
    out = _gather_reduce(P.reshape(-1), idx_r, bvec)
    return out.reshape(_B, 1)


# speedup vs baseline: 8.0328x; 2.3447x over previous
import functools

import jax
import jax.numpy as jnp
from jax import lax
from jax.experimental import pallas as pl
from jax.experimental.pallas import tpu as pltpu
from jax.experimental.pallas import tpu_sc as plsc

_VOCAB = 1000001
_VPAD = 1000008
_B = 16384
_L = 50
_D = 300
_C = 2 * _L
_CP = 128
_BM = 2048

_NC = 2
_NS = 16
_NW = _NC * _NS
_BPW = _B // _NW
_G = 128
_NG = _C * _BPW // _G
_K = 8


def _mm_body(t_ref, w_ref, o_ref):
    o_ref[...] = jax.lax.dot_general(
        t_ref[...], w_ref[...], (((0,), (0,)), ((), ())),
        preferred_element_type=jnp.float32)


def _score_table(table_t, wt):
    nb = pl.cdiv(_VPAD, _BM)
    return pl.pallas_call(
        _mm_body,
        grid=(nb,),
        in_specs=[
            pl.BlockSpec((_D, _BM), lambda i: (0, i)),
            pl.BlockSpec((_D, _CP), lambda i: (0, 0)),
        ],
        out_specs=pl.BlockSpec((_BM, _CP), lambda i: (i, 0)),
        out_shape=jax.ShapeDtypeStruct((_VPAD, _CP), jnp.float32),
    )(table_t, wt)


def _gather_reduce(p_flat, idx, bvec):
    mesh = plsc.VectorSubcoreMesh(core_axis_name="c", subcore_axis_name="s")

    @functools.partial(
        pl.kernel,
        mesh=mesh,
        out_type=jax.ShapeDtypeStruct((_B,), jnp.float32),
        scratch_types=[
            pltpu.VMEM((_NG, _G), jnp.int32),
            pltpu.VMEM((_NG * _G,), jnp.float32),
            pltpu.VMEM((16,), jnp.float32),
            pltpu.VMEM((_BPW,), jnp.float32),
            pltpu.SemaphoreType.DMA,
        ],
    )
    def k(p_hbm, idx_hbm, b_hbm, out_hbm, idx_v, g_v, b_v, out_v, sem):
        wid = lax.axis_index("s") * _NC + lax.axis_index("c")
        pltpu.sync_copy(idx_hbm.at[wid], idx_v)
        pltpu.sync_copy(b_hbm, b_v)

        def fire_group(gi, carry):
            cps = []
            for j in range(_K):
                row = gi * _K + j
                cps.append(pltpu.async_copy(
                    p_hbm.at[idx_v.at[row]],
                    g_v.at[pl.ds(row * _G, _G)],
                    sem,
                ))
            for cp in cps:
                cp.wait()
            return carry

        lax.fori_loop(0, _NG // _K, fire_group, 0)

        def col(c, carry):
            def red(l, acc):
                return acc + g_v[pl.ds(l * _BPW + c * 16, 16)]

            acc = lax.fori_loop(0, _C, red, b_v[...])
            out_v[pl.ds(c * 16, 16)] = 1.0 / (1.0 + jnp.exp(-acc))
            return carry

        lax.fori_loop(0, _BPW // 16, col, 0)
        pltpu.sync_copy(out_v, out_hbm.at[pl.ds(wid * _BPW, _BPW)])

    return k(p_flat, idx, bvec)


def kernel(input1, input2, table, W, b):
    wt = W.reshape(_C, _D).T.astype(jnp.float32)
    wt = jnp.pad(wt, ((0, 0), (0, _CP - _C)))
    table_t = table.astype(jnp.float32).T
    P = _score_table(table_t, wt)

    pos = jnp.arange(_L, dtype=jnp.int32)
    idx1 = input1 * _CP + pos[None, :]
    idx2 = input2 * _CP + (_L + pos)[None, :]
    idx_all = jnp.concatenate([idx1, idx2], axis=1)
    idx_r = (idx_all.reshape(_NW, _BPW, _C)
             .transpose(0, 2, 1)
             .reshape(_NW, _NG, _G))

    bvec = jnp.broadcast_to(b.astype(jnp.float32), (16,))
    out = _gather_reduce(P.reshape(-1), idx_r, bvec)
    return out.reshape(_B, 1)


# speedup vs baseline: 9.6899x; 1.2063x over previous
import functools

import jax
import jax.numpy as jnp
from jax import lax
from jax.experimental import pallas as pl
from jax.experimental.pallas import tpu as pltpu
from jax.experimental.pallas import tpu_sc as plsc

_VOCAB = 1000001
_VPAD = 1000008
_B = 16384
_L = 50
_D = 300
_C = 2 * _L
_CP = 128
_BM = 4096

_NC = 2
_NS = 16
_NW = _NC * _NS
_BPW = _B // _NW
_G = 128
_NG = _C * _BPW // _G
_K = 8


def _mm_body(t_ref, w_ref, o_ref):
    o_ref[...] = jax.lax.dot_general(
        t_ref[...], w_ref[...], (((0,), (0,)), ((), ())),
        preferred_element_type=jnp.float32)


def _score_table(table_t, wt):
    nb = pl.cdiv(_VPAD, _BM)
    return pl.pallas_call(
        _mm_body,
        grid=(nb,),
        in_specs=[
            pl.BlockSpec((_D, _BM), lambda i: (0, i)),
            pl.BlockSpec((_D, _CP), lambda i: (0, 0)),
        ],
        out_specs=pl.BlockSpec((_BM, _CP), lambda i: (i, 0)),
        out_shape=jax.ShapeDtypeStruct((_VPAD, _CP), jnp.float32),
    )(table_t, wt)


def _gather_reduce(p_flat, idx, bvec):
    mesh = plsc.VectorSubcoreMesh(core_axis_name="c", subcore_axis_name="s")

    @functools.partial(
        pl.kernel,
        mesh=mesh,
        out_type=jax.ShapeDtypeStruct((_B,), jnp.float32),
        scratch_types=[
            pltpu.VMEM((_NG, _G), jnp.int32),
            pltpu.VMEM((_NG * _G,), jnp.float32),
            pltpu.VMEM((16,), jnp.float32),
            pltpu.VMEM((_BPW,), jnp.float32),
            pltpu.SemaphoreType.DMA,
        ],
    )
    def k(p_hbm, idx_hbm, b_hbm, out_hbm, idx_v, g_v, b_v, out_v, sem):
        wid = lax.axis_index("s") * _NC + lax.axis_index("c")
        pltpu.sync_copy(idx_hbm.at[wid], idx_v)
        pltpu.sync_copy(b_hbm, b_v)

        def fire_group(gi, carry):
            cps = []
            for j in range(_K):
                row = gi * _K + j
                cps.append(pltpu.async_copy(
                    p_hbm.at[idx_v.at[row]],
                    g_v.at[pl.ds(row * _G, _G)],
                    sem,
                ))
            for cp in cps:
                cp.wait()
            return carry

        lax.fori_loop(0, _NG // _K, fire_group, 0)

        def col(c, carry):
            def red(l, acc):
                return acc + g_v[pl.ds(l * _BPW + c * 16, 16)]

            acc = lax.fori_loop(0, _C, red, b_v[...])
            out_v[pl.ds(c * 16, 16)] = 1.0 / (1.0 + jnp.exp(-acc))
            return carry

        lax.fori_loop(0, _BPW // 16, col, 0)
        pltpu.sync_copy(out_v, out_hbm.at[pl.ds(wid * _BPW, _BPW)])

    return k(p_flat, idx, bvec)


def kernel(input1, input2, table, W, b):
    wt = W.reshape(_C, _D).T.astype(jnp.float32)
    wt = jnp.pad(wt, ((0, 0), (0, _CP - _C)))
    table_t = table.astype(jnp.float32).T
    P = _score_table(table_t, wt)

    pos = jnp.arange(_L, dtype=jnp.int32)
    idx1 = input1 * _CP + pos[None, :]
    idx2 = input2 * _CP + (_L + pos)[None, :]
    idx_all = jnp.concatenate([idx1, idx2], axis=1)
    idx_r = (idx_all.reshape(_NW, _BPW, _C)
             .transpose(0, 2, 1)
             .reshape(_NW, _NG, _G))

    bvec = jnp.broadcast_to(b.astype(jnp.float32), (16,))
    out = _gather_reduce(P.reshape(-1), idx_r, bvec)
    return out.reshape(_B, 1)


# speedup vs baseline: 9.9057x; 1.0223x over previous
import functools

import jax
import jax.numpy as jnp
from jax import lax
from jax.experimental import pallas as pl
from jax.experimental.pallas import tpu as pltpu
from jax.experimental.pallas import tpu_sc as plsc

_VOCAB = 1000001
_VPAD = 1000008
_B = 16384
_L = 50
_D = 300
_C = 2 * _L
_CP = 128
_BM = 4096

_NC = 2
_NS = 16
_NW = _NC * _NS
_BPW = _B // _NW
_G = 128
_NG = _C * _BPW // _G
_K = 16


def _mm_body(t_ref, w_ref, o_ref):
    o_ref[...] = jax.lax.dot_general(
        t_ref[...], w_ref[...], (((0,), (0,)), ((), ())),
        preferred_element_type=jnp.float32)


def _score_table(table_t, wt):
    nb = pl.cdiv(_VPAD, _BM)
    return pl.pallas_call(
        _mm_body,
        grid=(nb,),
        in_specs=[
            pl.BlockSpec((_D, _BM), lambda i: (0, i)),
            pl.BlockSpec((_D, _CP), lambda i: (0, 0)),
        ],
        out_specs=pl.BlockSpec((_BM, _CP), lambda i: (i, 0)),
        out_shape=jax.ShapeDtypeStruct((_VPAD, _CP), jnp.float32),
    )(table_t, wt)


def _gather_reduce(p_flat, idx, bvec):
    mesh = plsc.VectorSubcoreMesh(core_axis_name="c", subcore_axis_name="s")

    @functools.partial(
        pl.kernel,
        mesh=mesh,
        out_type=jax.ShapeDtypeStruct((_B,), jnp.float32),
        scratch_types=[
            pltpu.VMEM((_NG, _G), jnp.int32),
            pltpu.VMEM((_NG * _G,), jnp.float32),
            pltpu.VMEM((16,), jnp.float32),
            pltpu.VMEM((_BPW,), jnp.float32),
            pltpu.SemaphoreType.DMA,
        ],
    )
    def k(p_hbm, idx_hbm, b_hbm, out_hbm, idx_v, g_v, b_v, out_v, sem):
        wid = lax.axis_index("s") * _NC + lax.axis_index("c")
        pltpu.sync_copy(idx_hbm.at[wid], idx_v)
        pltpu.sync_copy(b_hbm, b_v)

        def fire_group(gi, carry):
            cps = []
            for j in range(_K):
                row = gi * _K + j
                cps.append(pltpu.async_copy(
                    p_hbm.at[idx_v.at[row]],
                    g_v.at[pl.ds(row * _G, _G)],
                    sem,
                ))
            for cp in cps:
                cp.wait()
            return carry

        lax.fori_loop(0, _NG // _K, fire_group, 0)

        def col(c, carry):
            def red(l, acc):
                return acc + g_v[pl.ds(l * _BPW + c * 16, 16)]

            acc = lax.fori_loop(0, _C, red, b_v[...])
            out_v[pl.ds(c * 16, 16)] = 1.0 / (1.0 + jnp.exp(-acc))
            return carry

        lax.fori_loop(0, _BPW // 16, col, 0)
        pltpu.sync_copy(out_v, out_hbm.at[pl.ds(wid * _BPW, _BPW)])

    return k(p_flat, idx, bvec)


def kernel(input1, input2, table, W, b):
    wt = W.reshape(_C, _D).T.astype(jnp.float32)
    wt = jnp.pad(wt, ((0, 0), (0, _CP - _C)))
    table_t = table.astype(jnp.float32).T
    P = _score_table(table_t, wt)

    pos = jnp.arange(_L, dtype=jnp.int32)
    idx1 = input1 * _CP + pos[None, :]
    idx2 = input2 * _CP + (_L + pos)[None, :]
    idx_all = jnp.concatenate([idx1, idx2], axis=1)
    idx_r = (idx_all.reshape(_NW, _BPW, _C)
             .transpose(0, 2, 1)
             .reshape(_NW, _NG, _G))

    bvec = jnp.broadcast_to(b.astype(jnp.float32), (16,))
    out = _gather_reduce(P.reshape(-1), idx_r, bvec)
    return out.reshape(_B, 1)
